# baseline (speedup 1.0000x reference)
import jax
import jax.numpy as jnp
from jax import lax
from jax.experimental import pallas as pl
from jax.experimental.pallas import tpu as pltpu
from jax.experimental.pallas import tpu_sc as plsc

GX = 512
GY = 512
F = 64
HALF_F = F // 2
B = 16384
N = 200
P = B * N
NC, NS = 2, 16
NW = NC * NS
CHUNK = 128
NCHUNK = (N * B // CHUNK) // NW
BBLK = B // CHUNK
L = 16
TGROUPS = CHUNK // L
HI_MASK = -65536


def _sc_body(pos, emb_flat, out, bufs, sems):
    wid = lax.axis_index("s") * NC + lax.axis_index("c")
    cbase = wid * NCHUNK
    sem_xy, sem_g, sem_o = sems
    emb = emb_flat

    def pos_off(c):
        n = c // BBLK
        bh = c % BBLK
        return (n * B + bh * CHUNK) * 2

    def fire_xy(g, b):
        off = pos_off(cbase + g)
        pltpu.async_copy(pos.at[pl.ds(off, 2 * CHUNK)], bufs[b][0], sem_xy[b])

    def wait_xy(b):
        pltpu.make_async_copy(
            pos.at[pl.ds(0, 2 * CHUNK)], bufs[b][0], sem_xy[b]).wait()

    def index_phase(b):
        slab_v, w_v, i_v = bufs[b][0], bufs[b][1], bufs[b][2]
        lanes = jnp.arange(L, dtype=jnp.int32) * 2
        for g in range(TGROUPS):
            ix = lanes + (2 * L * g)
            x = plsc.load_gather(slab_v, [ix]) * float(GX)
            y = plsc.load_gather(slab_v, [ix + 1]) * float(GY)
            x0 = x.astype(jnp.int32)
            y0 = y.astype(jnp.int32)
            dx = x - x0.astype(jnp.float32)
            dy = y - y0.astype(jnp.float32)
            ex = 1.0 - dx
            ey = 1.0 - dy
            sl = pl.ds(g * L, L)
            w_v[0][sl] = ex * ey
            w_v[1][sl] = ex * dy
            w_v[2][sl] = dx * ey
            w_v[3][sl] = dx * dy
            x0c = jnp.clip(x0, 0, GX - 1)
            x1c = jnp.clip(x0 + 1, 0, GX - 1)
            y0c = jnp.clip(y0, 0, GY - 1)
            y1c = jnp.clip(y0 + 1, 0, GY - 1)
            r0 = x0c * GY
            r1 = x1c * GY
            i_v[0][sl] = r0 + y0c
            i_v[1][sl] = r0 + y1c
            i_v[2][sl] = r1 + y0c
            i_v[3][sl] = r1 + y1c

    def fire_gathers(b):
        i_v, r_v = bufs[b][2], bufs[b][3]
        for k in range(4):
            pltpu.async_copy(emb.at[i_v[k]], r_v[k], sem_g[b])

    def wait_gathers(b):
        i_v, r_v = bufs[b][2], bufs[b][3]
        for k in range(4):
            pltpu.make_async_copy(emb.at[i_v[k]], r_v[k], sem_g[b]).wait()

    def combine(b):
        w_v, r_v, out_v = bufs[b][1], bufs[b][3], bufs[b][4]
        for t in range(TGROUPS):
            psl = pl.ds(t * L, L)
            wts = [w_v[k][psl] for k in range(4)]
            pts = jnp.arange(L, dtype=jnp.int32) + (t * L)

            @plsc.parallel_loop(0, HALF_F, step=1, unroll=2)
            def word_body(j):
                jj = jnp.full((L,), j, dtype=jnp.int32)
                acc_lo = None
                acc_hi = None
                for k in range(4):
                    w = plsc.load_gather(r_v[k], [pts, jj])
                    flo = plsc.bitcast(w << 16, jnp.float32) * wts[k]
                    fhi = plsc.bitcast(w & HI_MASK, jnp.float32) * wts[k]
                    acc_lo = flo if acc_lo is None else acc_lo + flo
                    acc_hi = fhi if acc_hi is None else acc_hi + fhi
                fh = j >> 3
                fl = j & 7
                out_v[fh, pl.ds(fl * CHUNK + t * L, L)] = acc_lo
                out_v[fh + 4, pl.ds(fl * CHUNK + t * L, L)] = acc_hi

    def fire_out(g, b):
        c = cbase + g
        n = c // BBLK
        bh = c % BBLK
        out_v = bufs[b][4]
        for fh in range(8):
            pltpu.async_copy(
                out_v.at[fh], out.at[n, fh, bh], sem_o[b])

    def wait_out(b):
        out_v = bufs[b][4]
        for fh in range(8):
            pltpu.make_async_copy(
                out_v.at[fh], out.at[0, fh, 0], sem_o[b]).wait()

    def step(g, b, first, last, prefetch=True):
        nb = 1 - b
        if not last:
            wait_xy(nb)
            index_phase(nb)
            fire_gathers(nb)
        if prefetch:
            fire_xy(g + 2, b)
        wait_gathers(b)
        combine(b)
        if not first:
            wait_out(b)
        fire_out(g, b)

    fire_xy(0, 0)
    fire_xy(1, 1)
    wait_xy(0)
    index_phase(0)
    fire_gathers(0)

    step(0, 0, first=True, last=False)
    step(1, 1, first=True, last=False)

    def pair(i, _):
        g = 2 + 2 * i
        step(g, 0, first=False, last=False)
        step(g + 1, 1, first=False, last=False)
        return 0

    lax.fori_loop(0, (NCHUNK - 4) // 2, pair, 0)

    step(NCHUNK - 2, 0, first=False, last=False, prefetch=False)
    step(NCHUNK - 1, 1, first=False, last=True, prefetch=False)
    wait_out(0)
    wait_out(1)


def _buf_spec():
    return (
        pltpu.VMEM((2 * CHUNK,), jnp.float32),
        tuple(pltpu.VMEM((CHUNK,), jnp.float32) for _ in range(4)),
        tuple(pltpu.VMEM((CHUNK,), jnp.int32) for _ in range(4)),
        tuple(pltpu.VMEM((CHUNK, HALF_F), jnp.int32) for _ in range(4)),
        pltpu.VMEM((8, 8 * CHUNK), jnp.float32),
    )


@jax.jit
def _bilinear_sc(pos_t, emb_packed_flat):
    mesh = plsc.VectorSubcoreMesh(
        core_axis_name="c", subcore_axis_name="s",
        num_cores=NC, num_subcores=NS)
    f = pl.kernel(
        _sc_body,
        out_type=jax.ShapeDtypeStruct((N, 8, BBLK, 8 * CHUNK), jnp.float32),
        mesh=mesh,
        scratch_types=[
            (_buf_spec(), _buf_spec()),
            (
                (pltpu.SemaphoreType.DMA, pltpu.SemaphoreType.DMA),
                (pltpu.SemaphoreType.DMA, pltpu.SemaphoreType.DMA),
                (pltpu.SemaphoreType.DMA, pltpu.SemaphoreType.DMA),
            ),
        ],
        compiler_params=pltpu.CompilerParams(
            use_tc_tiling_on_sc=False, needs_layout_passes=False),
    )
    return f(pos_t, emb_packed_flat)


def kernel(positions, embeddings):
    pos_t = positions.transpose(1, 0, 2).reshape(2 * P)
    e = embeddings.reshape(GX * GY, F).astype(jnp.bfloat16)
    lo = lax.bitcast_convert_type(e[:, :HALF_F], jnp.uint16).astype(jnp.uint32)
    hi = lax.bitcast_convert_type(e[:, HALF_F:], jnp.uint16).astype(jnp.uint32)
    packed = lax.bitcast_convert_type(lo | (hi << 16), jnp.int32)
    x = _bilinear_sc(pos_t, packed)
    x = x.reshape(N, 8, BBLK, 8, CHUNK)
    return x.transpose(2, 4, 0, 1, 3).reshape(B, N, F)


# speedup vs baseline: 1.7281x; 1.7281x over previous
import jax
import jax.numpy as jnp
from jax import lax
from jax.experimental import pallas as pl
from jax.experimental.pallas import tpu as pltpu
from jax.experimental.pallas import tpu_sc as plsc

GX = 512
GY = 512
F = 64
HALF_F = F // 2
ROWW = HALF_F + 1
B = 16384
N = 200
P = B * N
NC, NS = 2, 16
NW = NC * NS
CHUNK = 128
NCHUNK = (N * B // CHUNK) // NW
BBLK = B // CHUNK
L = 16
TGROUPS = CHUNK // L
HI_MASK = -65536


def _sc_body(pos, emb_flat, out, bufs, sems):
    wid = lax.axis_index("s") * NC + lax.axis_index("c")
    cbase = wid * NCHUNK
    sem_xy, sem_g, sem_o = sems
    emb = emb_flat

    def pos_off(c):
        n = c // BBLK
        bh = c % BBLK
        return (n * B + bh * CHUNK) * 2

    def fire_xy(g, b):
        off = pos_off(cbase + g)
        pltpu.async_copy(pos.at[pl.ds(off, 2 * CHUNK)], bufs[b][0], sem_xy[b])

    def wait_xy(b):
        pltpu.make_async_copy(
            pos.at[pl.ds(0, 2 * CHUNK)], bufs[b][0], sem_xy[b]).wait()

    def index_phase(b):
        slab_v, w_v, i_v = bufs[b][0], bufs[b][1], bufs[b][2]
        lanes = jnp.arange(L, dtype=jnp.int32) * 2
        for g in range(TGROUPS):
            ix = lanes + (2 * L * g)
            x = plsc.load_gather(slab_v, [ix]) * float(GX)
            y = plsc.load_gather(slab_v, [ix + 1]) * float(GY)
            x0 = x.astype(jnp.int32)
            y0 = y.astype(jnp.int32)
            dx = x - x0.astype(jnp.float32)
            dy = y - y0.astype(jnp.float32)
            ex = 1.0 - dx
            ey = 1.0 - dy
            sl = pl.ds(g * L, L)
            w_v[0][sl] = ex * ey
            w_v[1][sl] = ex * dy
            w_v[2][sl] = dx * ey
            w_v[3][sl] = dx * dy
            x0c = jnp.clip(x0, 0, GX - 1)
            x1c = jnp.clip(x0 + 1, 0, GX - 1)
            y0c = jnp.clip(y0, 0, GY - 1)
            y1c = jnp.clip(y0 + 1, 0, GY - 1)
            r0 = x0c * GY
            r1 = x1c * GY
            i_v[0][sl] = r0 + y0c
            i_v[1][sl] = r0 + y1c
            i_v[2][sl] = r1 + y0c
            i_v[3][sl] = r1 + y1c

    def fire_gathers(b):
        i_v, r_v = bufs[b][2], bufs[b][3]
        for k in range(4):
            pltpu.async_copy(emb.at[i_v[k]], r_v[k], sem_g[b])

    def wait_gathers(b):
        i_v, r_v = bufs[b][2], bufs[b][3]
        for k in range(4):
            pltpu.make_async_copy(emb.at[i_v[k]], r_v[k], sem_g[b]).wait()

    def combine(b):
        w_v, r_v, out_v = bufs[b][1], bufs[b][3], bufs[b][4]
        for t in range(TGROUPS):
            psl = pl.ds(t * L, L)
            wts = [w_v[k][psl] for k in range(4)]
            pts = jnp.arange(L, dtype=jnp.int32) + (t * L)

            @plsc.parallel_loop(0, HALF_F, step=1, unroll=2)
            def word_body(j):
                jj = jnp.full((L,), j, dtype=jnp.int32)
                acc_lo = None
                acc_hi = None
                for k in range(4):
                    w = plsc.load_gather(r_v[k], [pts, jj])
                    flo = plsc.bitcast(w << 16, jnp.float32) * wts[k]
                    fhi = plsc.bitcast(w & HI_MASK, jnp.float32) * wts[k]
                    acc_lo = flo if acc_lo is None else acc_lo + flo
                    acc_hi = fhi if acc_hi is None else acc_hi + fhi
                fh = j >> 3
                fl = j & 7
                out_v[fh, pl.ds(fl * CHUNK + t * L, L)] = acc_lo
                out_v[fh + 4, pl.ds(fl * CHUNK + t * L, L)] = acc_hi

    def fire_out(g, b):
        c = cbase + g
        n = c // BBLK
        bh = c % BBLK
        out_v = bufs[b][4]
        for fh in range(8):
            pltpu.async_copy(
                out_v.at[fh], out.at[n, fh, bh], sem_o[b])

    def wait_out(b):
        out_v = bufs[b][4]
        for fh in range(8):
            pltpu.make_async_copy(
                out_v.at[fh], out.at[0, fh, 0], sem_o[b]).wait()

    def step(g, b, first, last, prefetch=True):
        nb = 1 - b
        if not last:
            wait_xy(nb)
            index_phase(nb)
            fire_gathers(nb)
        if prefetch:
            fire_xy(g + 2, b)
        wait_gathers(b)
        combine(b)
        if not first:
            wait_out(b)
        fire_out(g, b)

    fire_xy(0, 0)
    fire_xy(1, 1)
    wait_xy(0)
    index_phase(0)
    fire_gathers(0)

    step(0, 0, first=True, last=False)
    step(1, 1, first=True, last=False)

    def pair(i, _):
        g = 2 + 2 * i
        step(g, 0, first=False, last=False)
        step(g + 1, 1, first=False, last=False)
        return 0

    lax.fori_loop(0, (NCHUNK - 4) // 2, pair, 0)

    step(NCHUNK - 2, 0, first=False, last=False, prefetch=False)
    step(NCHUNK - 1, 1, first=False, last=True, prefetch=False)
    wait_out(0)
    wait_out(1)


def _buf_spec():
    return (
        pltpu.VMEM((2 * CHUNK,), jnp.float32),
        tuple(pltpu.VMEM((CHUNK,), jnp.float32) for _ in range(4)),
        tuple(pltpu.VMEM((CHUNK,), jnp.int32) for _ in range(4)),
        tuple(pltpu.VMEM((CHUNK, ROWW), jnp.int32) for _ in range(4)),
        pltpu.VMEM((8, 8 * CHUNK), jnp.float32),
    )


@jax.jit
def _bilinear_sc(pos_t, emb_packed_flat):
    mesh = plsc.VectorSubcoreMesh(
        core_axis_name="c", subcore_axis_name="s",
        num_cores=NC, num_subcores=NS)
    f = pl.kernel(
        _sc_body,
        out_type=jax.ShapeDtypeStruct((N, 8, BBLK, 8 * CHUNK), jnp.float32),
        mesh=mesh,
        scratch_types=[
            (_buf_spec(), _buf_spec()),
            (
                (pltpu.SemaphoreType.DMA, pltpu.SemaphoreType.DMA),
                (pltpu.SemaphoreType.DMA, pltpu.SemaphoreType.DMA),
                (pltpu.SemaphoreType.DMA, pltpu.SemaphoreType.DMA),
            ),
        ],
        compiler_params=pltpu.CompilerParams(
            use_tc_tiling_on_sc=False, needs_layout_passes=False),
    )
    return f(pos_t, emb_packed_flat)


def kernel(positions, embeddings):
    pos_t = positions.transpose(1, 0, 2).reshape(2 * P)
    e = embeddings.reshape(GX * GY, F).astype(jnp.bfloat16)
    lo = lax.bitcast_convert_type(e[:, :HALF_F], jnp.uint16).astype(jnp.uint32)
    hi = lax.bitcast_convert_type(e[:, HALF_F:], jnp.uint16).astype(jnp.uint32)
    packed = lax.bitcast_convert_type(lo | (hi << 16), jnp.int32)
    packed = jnp.pad(packed, ((0, 0), (0, ROWW - HALF_F)))
    x = _bilinear_sc(pos_t, packed)
    x = x.reshape(N, 8, BBLK, 8, CHUNK)
    return x.transpose(2, 4, 0, 1, 3).reshape(B, N, F)
